# baseline (speedup 1.0000x reference)
import jax
import jax.numpy as jnp
from jax.experimental import pallas as pl
from jax.experimental.pallas import tpu as pltpu

_LVL_HW = [(128, 128), (64, 64), (32, 32), (16, 16), (8, 8)]
_B = 2
_C = 16
_NLVL = 5
_X1 = 13.815509557963775

_f32 = jnp.float32
_bf16 = jnp.bfloat16

_TCLS, _TBB, _TANG, _TCTR = 0, 5, 10, 15
_SCLS, _SBB, _SANG, _SCTR = 20, 25, 30, 35


def _bce_logits(x, t):
    xc = jnp.clip(x, x.dtype.type(-_X1), x.dtype.type(_X1))
    sp = jnp.log(1 + jnp.exp(xc))
    return sp - t * xc


def _mxu_sum(a):
    b, c, h, w = a.shape
    a2 = a.reshape(b * c * h, w)
    ones = jnp.ones((1, b * c * h), _bf16)
    r = jax.lax.dot_general(ones, a2, (((1,), (0,)), ((), ())),
                            preferred_element_type=_f32)
    return jnp.sum(r)


def _smooth_l1(x, t):
    d = jnp.abs(x - t)
    m = jnp.minimum(d, d.dtype.type(1.0))
    return m * (d - d.dtype.type(0.5) * m)


def _loss_body(*refs):
    hbm = refs[:40]
    out_ref = refs[40]
    buf = refs[41:81]
    mask_refs = refs[81:86]
    sem = refs[86:126]

    copies = {}

    def start(i):
        cp = pltpu.make_async_copy(hbm[i], buf[i], sem[i])
        cp.start()
        copies[i] = cp

    for l in range(_NLVL):
        start(_TCLS + l)
        start(_TCTR + l)
    for l in range(_NLVL):
        for base in (_SCLS, _TBB, _SBB, _TANG, _SANG, _SCTR):
            start(base + l)

    tcv = []
    mv = []
    sig_tctr = []
    for l in range(_NLVL):
        copies[_TCLS + l].wait()
        copies[_TCTR + l].wait()
        tc = buf[_TCLS + l][...]
        tcv.append(tc)
        mx = jnp.max(tc, axis=1, keepdims=True)
        ct = buf[_TCTR + l][...]
        st = 0.5 * jnp.tanh(0.5 * ct) + 0.5
        mv.append((0.5 * jnp.tanh(0.5 * mx) + 0.5) * st)
        sig_tctr.append(st)

    num_cand = _f32(0.0)
    s1 = _f32(0.0)
    for m in mv:
        cf = (m >= 0.1).astype(_f32)
        num_cand = num_cand + jnp.sum(cf)
        s1 = s1 + jnp.sum(m * cf)
    cand_mean = s1 / num_cand
    s2 = _f32(0.0)
    for m in mv:
        cf = (m >= 0.1).astype(_f32)
        d = m - cand_mean
        s2 = s2 + jnp.sum(d * d * cf)
    cand_var = s2 / (num_cand - 1.0)
    pos_thresh = jnp.minimum(cand_mean + jnp.sqrt(cand_var), _f32(0.4))
    has_cand = num_cand > 0.0

    num_pos0 = _f32(0.0)
    for m in mv:
        p0 = jnp.logical_and(m >= pos_thresh, has_cand)
        num_pos0 = num_pos0 + jnp.sum(p0.astype(_f32))
    use_topk = num_pos0 == 0.0
    num_pos = jnp.where(use_topk, _f32(10.0), num_pos0)

    @pl.when(use_topk)
    def _topk():
        iotas = []
        for l in range(_NLVL):
            h, w = _LVL_HW[l]
            shp = (_B, 1, h, w)
            i0 = jax.lax.broadcasted_iota(jnp.int32, shp, 0)
            i2 = jax.lax.broadcasted_iota(jnp.int32, shp, 2)
            i3 = jax.lax.broadcasted_iota(jnp.int32, shp, 3)
            iotas.append((i0 * h + i2) * w + i3)
        work = list(mv)
        for _ in range(10):
            mcur = _f32(-1.0)
            for wv in work:
                mcur = jnp.maximum(mcur, jnp.max(wv))
            taken = jnp.bool_(False)
            new_work = []
            for l, wv in enumerate(work):
                eq = wv == mcur
                has = jnp.any(eq)
                do = jnp.logical_and(has, jnp.logical_not(taken))
                fi = jnp.min(jnp.where(eq, iotas[l], jnp.int32(2**30)))
                kill = jnp.logical_and(do, iotas[l] == fi)
                new_work.append(jnp.where(kill, _f32(-1.0), wv))
                taken = jnp.logical_or(taken, has)
            work = new_work
        for l in range(_NLVL):
            mask_refs[l][...] = (work[l] < 0.0).astype(_f32)

    total = _f32(0.0)
    for l in range(_NLVL):
        m = mv[l]
        candf = (m >= 0.1).astype(_f32)
        p0f = jnp.logical_and(m >= pos_thresh, has_cand).astype(_f32)
        posf = jnp.where(use_topk, mask_refs[l][...], p0f)
        hnf = candf * (1.0 - p0f)
        keepf = jnp.maximum(posf, hnf)
        pos_b = posf.astype(_bf16)

        copies[_SCLS + l].wait()
        x = buf[_SCLS + l][...].astype(_bf16)
        yb = tcv[l].astype(_bf16)
        sig_y = _bf16(0.5) * jnp.tanh(_bf16(0.5) * yb) + _bf16(0.5)
        t = keepf.astype(_bf16) * sig_y
        total = total + _mxu_sum(_bce_logits(x, t))

        for base in (_TBB, _SBB, _TANG, _SANG):
            copies[base + l].wait()
        lb = jnp.sum(
            _smooth_l1(buf[_SBB + l][...].astype(_bf16),
                       buf[_TBB + l][...].astype(_bf16)),
            axis=1,
            keepdims=True,
        )
        la = _smooth_l1(buf[_SANG + l][...].astype(_bf16),
                        buf[_TANG + l][...].astype(_bf16))
        total = total + _mxu_sum((lb + la) * pos_b)

        copies[_SCTR + l].wait()
        xi = buf[_SCTR + l][...].astype(_bf16)
        ti = sig_tctr[l].astype(_bf16)
        total = total + _mxu_sum(_bce_logits(xi, ti) * pos_b)

    out_ref[0, 0] = total / num_pos


def _in_shapes():
    shapes = []
    for ch in (_C, 4, 1, 1):
        for h, w in _LVL_HW:
            shapes.append((_B, ch, h, w))
    return shapes + shapes


def _run(args, interpret=False):
    shapes = _in_shapes()
    scratch = [pltpu.VMEM(s, _f32) for s in shapes]
    scratch += [pltpu.VMEM((_B, 1, h, w), _f32) for h, w in _LVL_HW]
    scratch += [pltpu.SemaphoreType.DMA] * 40
    out = pl.pallas_call(
        _loss_body,
        out_shape=jax.ShapeDtypeStruct((1, 1), jnp.float32),
        in_specs=[pl.BlockSpec(memory_space=pltpu.MemorySpace.HBM)] * 40,
        out_specs=pl.BlockSpec(memory_space=pltpu.SMEM),
        scratch_shapes=scratch,
        interpret=interpret,
    )(*args)
    return out.reshape(())


def kernel(t_cls_0, t_bbox_0, t_angle_0, t_ctr_0, t_cls_1, t_bbox_1, t_angle_1, t_ctr_1, t_cls_2, t_bbox_2, t_angle_2, t_ctr_2, t_cls_3, t_bbox_3, t_angle_3, t_ctr_3, t_cls_4, t_bbox_4, t_angle_4, t_ctr_4, s_cls_0, s_bbox_0, s_angle_0, s_ctr_0, s_cls_1, s_bbox_1, s_angle_1, s_ctr_1, s_cls_2, s_bbox_2, s_angle_2, s_ctr_2, s_cls_3, s_bbox_3, s_angle_3, s_ctr_3, s_cls_4, s_bbox_4, s_angle_4, s_ctr_4):
    loc = dict(locals())
    args = []
    for pre in ("t", "s"):
        for kind in ("cls", "bbox", "angle", "ctr"):
            for l in range(_NLVL):
                args.append(loc[f"{pre}_{kind}_{l}"])
    return _run(args)


# speedup vs baseline: 1.2223x; 1.2223x over previous
import jax
import jax.numpy as jnp
from jax.experimental import pallas as pl
from jax.experimental.pallas import tpu as pltpu

_LVL_HW = [(128, 128), (64, 64), (32, 32), (16, 16), (8, 8)]
_B = 2
_C = 16
_NLVL = 5
_X1 = 13.815509557963775

_f32 = jnp.float32
_bf16 = jnp.bfloat16

_TCLS, _TBB, _TANG, _TCTR = 0, 5, 10, 15
_SCLS, _SBB, _SANG, _SCTR = 20, 25, 30, 35


def _bce_logits(x, t):
    xc = jnp.clip(x, x.dtype.type(-_X1), x.dtype.type(_X1))
    sp = jnp.log(1 + jnp.exp(xc))
    return sp - t * xc


def _mxu_sum(a):
    b, c, h, w = a.shape
    a2 = a.reshape(b * c * h, w)
    ones = jnp.ones((1, b * c * h), _bf16)
    r = jax.lax.dot_general(ones, a2, (((1,), (0,)), ((), ())),
                            preferred_element_type=_f32)
    return jnp.sum(r)


def _smooth_l1(x, t):
    d = jnp.abs(x - t)
    m = jnp.minimum(d, d.dtype.type(1.0))
    return m * (d - d.dtype.type(0.5) * m)


def _loss_body(*refs):
    hbm = refs[:40]
    out_ref = refs[40]
    buf = refs[41:81]
    mask_refs = refs[81:86]
    sem = refs[86:126]

    copies = {}

    def start(i):
        cp = pltpu.make_async_copy(hbm[i], buf[i], sem[i])
        cp.start()
        copies[i] = cp

    for l in range(_NLVL):
        start(_TCLS + l)
        start(_TCTR + l)
    for l in range(_NLVL):
        for base in (_SCLS, _TBB, _SBB, _TANG, _SANG, _SCTR):
            start(base + l)

    tcv = []
    mv = []
    sig_tctr = []
    for l in range(_NLVL):
        copies[_TCLS + l].wait()
        copies[_TCTR + l].wait()
        tc = buf[_TCLS + l][...]
        tcv.append(tc)
        mx = jnp.max(tc, axis=1, keepdims=True)
        ct = buf[_TCTR + l][...]
        st = 0.5 * jnp.tanh(0.5 * ct) + 0.5
        mv.append((0.5 * jnp.tanh(0.5 * mx) + 0.5) * st)
        sig_tctr.append(st)

    num_cand = _f32(0.0)
    s1 = _f32(0.0)
    for m in mv:
        cf = (m >= 0.1).astype(_f32)
        num_cand = num_cand + jnp.sum(cf)
        s1 = s1 + jnp.sum(m * cf)
    cand_mean = s1 / num_cand
    s2 = _f32(0.0)
    for m in mv:
        cf = (m >= 0.1).astype(_f32)
        d = m - cand_mean
        s2 = s2 + jnp.sum(d * d * cf)
    cand_var = s2 / (num_cand - 1.0)
    pos_thresh = jnp.minimum(cand_mean + jnp.sqrt(cand_var), _f32(0.4))
    has_cand = num_cand > 0.0

    num_pos0 = _f32(0.0)
    for m in mv:
        p0 = jnp.logical_and(m >= pos_thresh, has_cand)
        num_pos0 = num_pos0 + jnp.sum(p0.astype(_f32))
    use_topk = num_pos0 == 0.0
    num_pos = jnp.where(use_topk, _f32(10.0), num_pos0)

    @pl.when(use_topk)
    def _topk():
        iotas = []
        for l in range(_NLVL):
            h, w = _LVL_HW[l]
            shp = (_B, 1, h, w)
            i0 = jax.lax.broadcasted_iota(jnp.int32, shp, 0)
            i2 = jax.lax.broadcasted_iota(jnp.int32, shp, 2)
            i3 = jax.lax.broadcasted_iota(jnp.int32, shp, 3)
            iotas.append((i0 * h + i2) * w + i3)
        work = list(mv)
        for _ in range(10):
            mcur = _f32(-1.0)
            for wv in work:
                mcur = jnp.maximum(mcur, jnp.max(wv))
            taken = jnp.bool_(False)
            new_work = []
            for l, wv in enumerate(work):
                eq = wv == mcur
                has = jnp.any(eq)
                do = jnp.logical_and(has, jnp.logical_not(taken))
                fi = jnp.min(jnp.where(eq, iotas[l], jnp.int32(2**30)))
                kill = jnp.logical_and(do, iotas[l] == fi)
                new_work.append(jnp.where(kill, _f32(-1.0), wv))
                taken = jnp.logical_or(taken, has)
            work = new_work
        for l in range(_NLVL):
            mask_refs[l][...] = (work[l] < 0.0).astype(_f32)

    for l in range(_NLVL):
        for base in (_SCLS, _TBB, _TANG, _SBB, _SANG, _SCTR):
            copies[base + l].wait()
    total = _f32(0.0)
    for l in range(_NLVL):
        m = mv[l]
        candf = (m >= 0.1).astype(_f32)
        p0f = jnp.logical_and(m >= pos_thresh, has_cand).astype(_f32)
        posf = jnp.where(use_topk, mask_refs[l][...], p0f)
        hnf = candf * (1.0 - p0f)
        keepf = jnp.maximum(posf, hnf)
        pos_b = posf.astype(_bf16)

        x = buf[_SCLS + l][...].astype(_bf16)
        yb = tcv[l].astype(_bf16)
        sig_y = _bf16(0.5) * jnp.tanh(_bf16(0.5) * yb) + _bf16(0.5)
        t = keepf.astype(_bf16) * sig_y
        total = total + _mxu_sum(_bce_logits(x, t))

        lb = jnp.sum(
            _smooth_l1(buf[_SBB + l][...].astype(_bf16),
                       buf[_TBB + l][...].astype(_bf16)),
            axis=1,
            keepdims=True,
        )
        la = _smooth_l1(buf[_SANG + l][...].astype(_bf16),
                        buf[_TANG + l][...].astype(_bf16))
        total = total + _mxu_sum((lb + la) * pos_b)

        xi = buf[_SCTR + l][...].astype(_bf16)
        ti = sig_tctr[l].astype(_bf16)
        total = total + _mxu_sum(_bce_logits(xi, ti) * pos_b)

    out_ref[0, 0] = total / num_pos


def _in_shapes():
    shapes = []
    for ch in (_C, 4, 1, 1):
        for h, w in _LVL_HW:
            shapes.append((_B, ch, h, w))
    return shapes + shapes


def _run(args, interpret=False):
    shapes = _in_shapes()
    scratch = [pltpu.VMEM(s, _f32) for s in shapes]
    scratch += [pltpu.VMEM((_B, 1, h, w), _f32) for h, w in _LVL_HW]
    scratch += [pltpu.SemaphoreType.DMA] * 40
    out = pl.pallas_call(
        _loss_body,
        out_shape=jax.ShapeDtypeStruct((1, 1), jnp.float32),
        in_specs=[pl.BlockSpec(memory_space=pltpu.MemorySpace.HBM)] * 40,
        out_specs=pl.BlockSpec(memory_space=pltpu.SMEM),
        scratch_shapes=scratch,
        interpret=interpret,
    )(*args)
    return out.reshape(())


def kernel(t_cls_0, t_bbox_0, t_angle_0, t_ctr_0, t_cls_1, t_bbox_1, t_angle_1, t_ctr_1, t_cls_2, t_bbox_2, t_angle_2, t_ctr_2, t_cls_3, t_bbox_3, t_angle_3, t_ctr_3, t_cls_4, t_bbox_4, t_angle_4, t_ctr_4, s_cls_0, s_bbox_0, s_angle_0, s_ctr_0, s_cls_1, s_bbox_1, s_angle_1, s_ctr_1, s_cls_2, s_bbox_2, s_angle_2, s_ctr_2, s_cls_3, s_bbox_3, s_angle_3, s_ctr_3, s_cls_4, s_bbox_4, s_angle_4, s_ctr_4):
    loc = dict(locals())
    args = []
    for pre in ("t", "s"):
        for kind in ("cls", "bbox", "angle", "ctr"):
            for l in range(_NLVL):
                args.append(loc[f"{pre}_{kind}_{l}"])
    return _run(args)


# speedup vs baseline: 1.2234x; 1.0009x over previous
import jax
import jax.numpy as jnp
from jax.experimental import pallas as pl
from jax.experimental.pallas import tpu as pltpu

_LVL_HW = [(128, 128), (64, 64), (32, 32), (16, 16), (8, 8)]
_B = 2
_C = 16
_NLVL = 5
_X1 = 13.815509557963775

_f32 = jnp.float32
_bf16 = jnp.bfloat16

_TCLS, _TBB, _TANG, _TCTR = 0, 5, 10, 15
_SCLS, _SBB, _SANG, _SCTR = 20, 25, 30, 35


def _bce_logits(x, t):
    xc = jnp.clip(x, x.dtype.type(-_X1), x.dtype.type(_X1))
    sp = jnp.log(1 + jnp.exp(xc))
    return sp - t * xc


def _mxu_sum(a):
    b, c, h, w = a.shape
    a2 = a.reshape(b * c * h, w)
    ones = jnp.ones((1, b * c * h), _bf16)
    r = jax.lax.dot_general(ones, a2, (((1,), (0,)), ((), ())),
                            preferred_element_type=_f32)
    return jnp.sum(r)


def _smooth_l1(x, t):
    d = jnp.abs(x - t)
    m = jnp.minimum(d, d.dtype.type(1.0))
    return m * (d - d.dtype.type(0.5) * m)


def _loss_body(*refs):
    hbm = refs[:40]
    out_ref = refs[40]
    buf = refs[41:81]
    mask_refs = refs[81:86]
    sem = refs[86:126]

    copies = {}

    def start(i):
        cp = pltpu.make_async_copy(hbm[i], buf[i], sem[i])
        cp.start()
        copies[i] = cp

    for l in range(_NLVL):
        start(_TCLS + l)
        start(_TCTR + l)
    for l in range(_NLVL):
        for base in (_SCLS, _TBB, _SBB, _TANG, _SANG, _SCTR):
            start(base + l)

    tcv = []
    mv = []
    sig_tctr = []
    for l in range(_NLVL):
        copies[_TCLS + l].wait()
        copies[_TCTR + l].wait()
    for l in range(_NLVL):
        tc = buf[_TCLS + l][...]
        tcv.append(tc)
        mx = jnp.max(tc, axis=1, keepdims=True)
        ct = buf[_TCTR + l][...]
        st = 0.5 * jnp.tanh(0.5 * ct) + 0.5
        mv.append((0.5 * jnp.tanh(0.5 * mx) + 0.5) * st)
        sig_tctr.append(st)

    num_cand = _f32(0.0)
    s1 = _f32(0.0)
    for m in mv:
        cf = (m >= 0.1).astype(_f32)
        num_cand = num_cand + jnp.sum(cf)
        s1 = s1 + jnp.sum(m * cf)
    cand_mean = s1 / num_cand
    s2 = _f32(0.0)
    for m in mv:
        cf = (m >= 0.1).astype(_f32)
        d = m - cand_mean
        s2 = s2 + jnp.sum(d * d * cf)
    cand_var = s2 / (num_cand - 1.0)
    pos_thresh = jnp.minimum(cand_mean + jnp.sqrt(cand_var), _f32(0.4))
    has_cand = num_cand > 0.0

    num_pos0 = _f32(0.0)
    for m in mv:
        p0 = jnp.logical_and(m >= pos_thresh, has_cand)
        num_pos0 = num_pos0 + jnp.sum(p0.astype(_f32))
    use_topk = num_pos0 == 0.0
    num_pos = jnp.where(use_topk, _f32(10.0), num_pos0)

    @pl.when(use_topk)
    def _topk():
        iotas = []
        for l in range(_NLVL):
            h, w = _LVL_HW[l]
            shp = (_B, 1, h, w)
            i0 = jax.lax.broadcasted_iota(jnp.int32, shp, 0)
            i2 = jax.lax.broadcasted_iota(jnp.int32, shp, 2)
            i3 = jax.lax.broadcasted_iota(jnp.int32, shp, 3)
            iotas.append((i0 * h + i2) * w + i3)
        work = list(mv)
        for _ in range(10):
            mcur = _f32(-1.0)
            for wv in work:
                mcur = jnp.maximum(mcur, jnp.max(wv))
            taken = jnp.bool_(False)
            new_work = []
            for l, wv in enumerate(work):
                eq = wv == mcur
                has = jnp.any(eq)
                do = jnp.logical_and(has, jnp.logical_not(taken))
                fi = jnp.min(jnp.where(eq, iotas[l], jnp.int32(2**30)))
                kill = jnp.logical_and(do, iotas[l] == fi)
                new_work.append(jnp.where(kill, _f32(-1.0), wv))
                taken = jnp.logical_or(taken, has)
            work = new_work
        for l in range(_NLVL):
            mask_refs[l][...] = (work[l] < 0.0).astype(_f32)

    for l in range(_NLVL):
        for base in (_SCLS, _TBB, _TANG, _SBB, _SANG, _SCTR):
            copies[base + l].wait()
    total = _f32(0.0)
    for l in range(_NLVL):
        m = mv[l]
        candf = (m >= 0.1).astype(_f32)
        p0f = jnp.logical_and(m >= pos_thresh, has_cand).astype(_f32)
        posf = jnp.where(use_topk, mask_refs[l][...], p0f)
        hnf = candf * (1.0 - p0f)
        keepf = jnp.maximum(posf, hnf)
        pos_b = posf.astype(_bf16)

        x = buf[_SCLS + l][...].astype(_bf16)
        yb = tcv[l].astype(_bf16)
        sig_y = _bf16(0.5) * jnp.tanh(_bf16(0.5) * yb) + _bf16(0.5)
        t = keepf.astype(_bf16) * sig_y
        total = total + _mxu_sum(_bce_logits(x, t))

        lb = jnp.sum(
            _smooth_l1(buf[_SBB + l][...].astype(_bf16),
                       buf[_TBB + l][...].astype(_bf16)),
            axis=1,
            keepdims=True,
        )
        la = _smooth_l1(buf[_SANG + l][...].astype(_bf16),
                        buf[_TANG + l][...].astype(_bf16))
        total = total + _mxu_sum((lb + la) * pos_b)

        xi = buf[_SCTR + l][...].astype(_bf16)
        ti = sig_tctr[l].astype(_bf16)
        total = total + _mxu_sum(_bce_logits(xi, ti) * pos_b)

    out_ref[0, 0] = total / num_pos


def _in_shapes():
    shapes = []
    for ch in (_C, 4, 1, 1):
        for h, w in _LVL_HW:
            shapes.append((_B, ch, h, w))
    return shapes + shapes


def _run(args, interpret=False):
    shapes = _in_shapes()
    scratch = [pltpu.VMEM(s, _f32) for s in shapes]
    scratch += [pltpu.VMEM((_B, 1, h, w), _f32) for h, w in _LVL_HW]
    scratch += [pltpu.SemaphoreType.DMA] * 40
    out = pl.pallas_call(
        _loss_body,
        out_shape=jax.ShapeDtypeStruct((1, 1), jnp.float32),
        in_specs=[pl.BlockSpec(memory_space=pltpu.MemorySpace.HBM)] * 40,
        out_specs=pl.BlockSpec(memory_space=pltpu.SMEM),
        scratch_shapes=scratch,
        interpret=interpret,
    )(*args)
    return out.reshape(())


def kernel(t_cls_0, t_bbox_0, t_angle_0, t_ctr_0, t_cls_1, t_bbox_1, t_angle_1, t_ctr_1, t_cls_2, t_bbox_2, t_angle_2, t_ctr_2, t_cls_3, t_bbox_3, t_angle_3, t_ctr_3, t_cls_4, t_bbox_4, t_angle_4, t_ctr_4, s_cls_0, s_bbox_0, s_angle_0, s_ctr_0, s_cls_1, s_bbox_1, s_angle_1, s_ctr_1, s_cls_2, s_bbox_2, s_angle_2, s_ctr_2, s_cls_3, s_bbox_3, s_angle_3, s_ctr_3, s_cls_4, s_bbox_4, s_angle_4, s_ctr_4):
    loc = dict(locals())
    args = []
    for pre in ("t", "s"):
        for kind in ("cls", "bbox", "angle", "ctr"):
            for l in range(_NLVL):
                args.append(loc[f"{pre}_{kind}_{l}"])
    return _run(args)
